# baseline (speedup 1.0000x reference)
import functools

import jax
import jax.numpy as jnp
from jax.experimental import pallas as pl
from jax.experimental.pallas import tpu as pltpu

_NUM_CORES = 2
_CHUNK_ROWS = 256
_NBUF = 4


def _dma_pipe(emb_ref, out_ref, scratch, in_sems, out_sems, *, ch, nchunks):
    base = pl.program_id(0) * ch * nchunks

    def in_copy(c, slot):
        return pltpu.make_async_copy(
            emb_ref.at[pl.ds(base + c * ch, ch)], scratch.at[slot],
            in_sems.at[slot])

    def out_copy(c, slot):
        return pltpu.make_async_copy(
            scratch.at[slot], out_ref.at[pl.ds(base + c * ch, ch)],
            out_sems.at[slot])

    nbuf = min(_NBUF, nchunks)
    for c in range(nbuf):
        in_copy(c, c).start()
    for c in range(nchunks):
        slot = c % nbuf
        in_copy(c, slot).wait()
        out_copy(c, slot).start()
        nxt = c + nbuf
        if nxt < nchunks:
            out_copy(c, slot).wait()
            in_copy(nxt, slot).start()
    for c in range(max(0, nchunks - nbuf), nchunks):
        out_copy(c, c % nbuf).wait()


def kernel(inputs, embeddings):
    seq_len = inputs.shape[1]
    emb_dim = embeddings.shape[1]
    table = embeddings[:seq_len, :]
    rows_per_core = seq_len // _NUM_CORES
    ch = min(_CHUNK_ROWS, rows_per_core)
    nchunks = rows_per_core // ch
    out = pl.pallas_call(
        functools.partial(_dma_pipe, ch=ch, nchunks=nchunks),
        grid=(_NUM_CORES,),
        in_specs=[pl.BlockSpec(memory_space=pl.ANY)],
        out_specs=pl.BlockSpec(memory_space=pl.ANY),
        out_shape=jax.ShapeDtypeStruct((seq_len, emb_dim), embeddings.dtype),
        scratch_shapes=[
            pltpu.VMEM((_NBUF, ch, emb_dim), embeddings.dtype),
            pltpu.SemaphoreType.DMA((_NBUF,)),
            pltpu.SemaphoreType.DMA((_NBUF,)),
        ],
        compiler_params=pltpu.CompilerParams(
            dimension_semantics=("parallel",),
        ),
    )(table)
    return out[None]


# speedup vs baseline: 1.4439x; 1.4439x over previous
import functools

import jax
import jax.numpy as jnp
from jax.experimental import pallas as pl
from jax.experimental.pallas import tpu as pltpu

_NUM_CORES = 2
_CHUNK_ROWS = 512


def _dma_pipe(emb_ref, out_ref, scratch, in_sems, out_sems, *, ch, nchunks):
    base = pl.program_id(0) * ch * nchunks

    def in_copy(c):
        return pltpu.make_async_copy(
            emb_ref.at[pl.ds(base + c * ch, ch)], scratch.at[c],
            in_sems.at[c])

    def out_copy(c):
        return pltpu.make_async_copy(
            scratch.at[c], out_ref.at[pl.ds(base + c * ch, ch)],
            out_sems.at[c])

    for c in range(nchunks):
        in_copy(c).start()
    for c in range(nchunks):
        in_copy(c).wait()
        out_copy(c).start()
    for c in range(nchunks):
        out_copy(c).wait()


def kernel(inputs, embeddings):
    seq_len = inputs.shape[1]
    emb_dim = embeddings.shape[1]
    table = embeddings[:seq_len, :]
    rows_per_core = seq_len // _NUM_CORES
    ch = min(_CHUNK_ROWS, rows_per_core)
    nchunks = rows_per_core // ch
    out = pl.pallas_call(
        functools.partial(_dma_pipe, ch=ch, nchunks=nchunks),
        grid=(_NUM_CORES,),
        in_specs=[pl.BlockSpec(memory_space=pl.ANY)],
        out_specs=pl.BlockSpec(memory_space=pl.ANY),
        out_shape=jax.ShapeDtypeStruct((seq_len, emb_dim), embeddings.dtype),
        scratch_shapes=[
            pltpu.VMEM((nchunks, ch, emb_dim), embeddings.dtype),
            pltpu.SemaphoreType.DMA((nchunks,)),
            pltpu.SemaphoreType.DMA((nchunks,)),
        ],
        compiler_params=pltpu.CompilerParams(
            dimension_semantics=("parallel",),
        ),
    )(table)
    return out[None]


# speedup vs baseline: 1.6209x; 1.1226x over previous
import jax
import jax.numpy as jnp
from jax.experimental import pallas as pl
from jax.experimental.pallas import tpu as pltpu

_BLOCK_ROWS = 2048


def _copy_block(emb_ref, out_ref):
    out_ref[...] = emb_ref[...]


def kernel(inputs, embeddings):
    seq_len = inputs.shape[1]
    emb_dim = embeddings.shape[1]
    table = embeddings[:seq_len, :]
    blk = min(_BLOCK_ROWS, seq_len)
    grid = (seq_len // blk,)
    out = pl.pallas_call(
        _copy_block,
        grid=grid,
        in_specs=[pl.BlockSpec((blk, emb_dim), lambda i: (i, 0))],
        out_specs=pl.BlockSpec((blk, emb_dim), lambda i: (i, 0)),
        out_shape=jax.ShapeDtypeStruct((seq_len, emb_dim), embeddings.dtype),
        compiler_params=pltpu.CompilerParams(
            dimension_semantics=("parallel",),
        ),
    )(table)
    return out[None]
